# baseline (speedup 1.0000x reference)
import functools

import jax
import jax.numpy as jnp
from jax import lax
from jax.experimental import pallas as pl
from jax.experimental.pallas import tpu as pltpu
from jax.experimental.pallas import tpu_sc as plsc

EMBED = 32
BATCH = 16384
NC = 2
NS = 16
NW = NC * NS
B_PER_W = BATCH // NW
CHUNK = 128
NCHUNK = B_PER_W // CHUNK


def _gather_body(uid_hbm, iid_hbm, ut_hbm, it_hbm, ue_hbm, ie_hbm,
                 uidx_v, iidx_v, urows_v, irows_v, sem):
    wid = lax.axis_index("s") * NC + lax.axis_index("c")
    base = wid * B_PER_W
    pltpu.sync_copy(uid_hbm.at[wid], uidx_v)
    pltpu.sync_copy(iid_hbm.at[wid], iidx_v)
    copies = []
    for j in range(NCHUNK):
        copies.append(pltpu.async_copy(
            ut_hbm.at[uidx_v.at[j]], urows_v.at[pl.ds(j * CHUNK, CHUNK)], sem))
    for j in range(NCHUNK):
        copies.append(pltpu.async_copy(
            it_hbm.at[iidx_v.at[j]], irows_v.at[pl.ds(j * CHUNK, CHUNK)], sem))
    for c in copies:
        c.wait()
    pltpu.sync_copy(urows_v, ue_hbm.at[pl.ds(base, B_PER_W)])
    pltpu.sync_copy(irows_v, ie_hbm.at[pl.ds(base, B_PER_W)])


@functools.cache
def _gather():
    return pl.kernel(
        _gather_body,
        out_type=(
            jax.ShapeDtypeStruct((BATCH, EMBED), jnp.float32),
            jax.ShapeDtypeStruct((BATCH, EMBED), jnp.float32),
        ),
        mesh=plsc.VectorSubcoreMesh(core_axis_name="c", subcore_axis_name="s",
                                    num_cores=NC, num_subcores=NS),
        scratch_types=[
            pltpu.VMEM((NCHUNK, CHUNK), jnp.int32),
            pltpu.VMEM((NCHUNK, CHUNK), jnp.int32),
            pltpu.VMEM((B_PER_W, EMBED), jnp.float32),
            pltpu.VMEM((B_PER_W, EMBED), jnp.float32),
            pltpu.SemaphoreType.DMA,
        ],
        compiler_params=pltpu.CompilerParams(use_tc_tiling_on_sc=False),
    )


BLK = 1024


def _mlp_body(xu_ref, xi_ref, w1u_ref, w1i_ref, b1_ref, w2_ref, b2_ref,
              w3_ref, b3_ref, out_ref):
    x1 = jnp.dot(xu_ref[...], w1u_ref[...], preferred_element_type=jnp.float32)
    x2 = jnp.dot(xi_ref[...], w1i_ref[...], preferred_element_type=jnp.float32)
    h = jnp.maximum(x1 + x2 + b1_ref[...], 0.0)
    h = jnp.maximum(
        jnp.dot(h, w2_ref[...], preferred_element_type=jnp.float32) + b2_ref[...],
        0.0)
    out_ref[...] = (
        jnp.dot(h, w3_ref[...], preferred_element_type=jnp.float32) + b3_ref[...])


def _mlp(xu, xi, w1u, w1i, b1, w2, b2, w3, b3):
    grid = (BATCH // BLK,)
    full = lambda shape: pl.BlockSpec(shape, lambda i: (0,) * len(shape))
    return pl.pallas_call(
        _mlp_body,
        grid=grid,
        in_specs=[
            pl.BlockSpec((BLK, EMBED), lambda i: (i, 0)),
            pl.BlockSpec((BLK, EMBED), lambda i: (i, 0)),
            full((EMBED, 256)),
            full((EMBED, 256)),
            full((1, 256)),
            full((256, 64)),
            full((1, 64)),
            full((64, 1)),
            full((1, 1)),
        ],
        out_specs=pl.BlockSpec((BLK, 1), lambda i: (i, 0)),
        out_shape=jax.ShapeDtypeStruct((BATCH, 1), jnp.float32),
    )(xu, xi, w1u, w1i, b1, w2, b2, w3, b3)


def kernel(user_id, item_id, user_table, item_table, W1, b1, W2, b2, W3, b3):
    uid = user_id.astype(jnp.int32).reshape(NW, NCHUNK, CHUNK)
    iid = item_id.astype(jnp.int32).reshape(NW, NCHUNK, CHUNK)
    ue, ie = _gather()(uid, iid, user_table, item_table)
    return _mlp(ue, ie, W1[:EMBED], W1[EMBED:], b1.reshape(1, 256),
                W2, b2.reshape(1, 64), W3, b3.reshape(1, 1))


# speedup vs baseline: 1.4823x; 1.4823x over previous
import functools

import jax
import jax.numpy as jnp
from jax import lax
from jax.experimental import pallas as pl
from jax.experimental.pallas import tpu as pltpu
from jax.experimental.pallas import tpu_sc as plsc

EMBED = 32
BATCH = 16384
NC = 2
NS = 16
NW = NC * NS
B_PER_W = BATCH // NW
CHUNK = 128
NCHUNK = B_PER_W // CHUNK
HALF = B_PER_W // 2


def _gather_body(idx_hbm, ut_hbm, it_hbm, ue_hbm, ie_hbm,
                 idx_v, urows_v, irows_v, semu, semi):
    wid = lax.axis_index("s") * NC + lax.axis_index("c")
    base = wid * B_PER_W
    pltpu.sync_copy(idx_hbm.at[wid], idx_v)
    lanes = lax.iota(jnp.int32, 16)

    vecs_per_half = HALF // 16
    for h in range(B_PER_W // HALF):
        def body(t, carry):
            g = h * vecs_per_half + t
            a = g // 8
            c = g % 8
            uvec = idx_v[a, pl.ds(c * 16, 16)]
            ivec = idx_v[NCHUNK + a, pl.ds(c * 16, 16)]
            for l in range(16):
                ui = jnp.sum(jnp.where(lanes == l, uvec, 0))
                ii = jnp.sum(jnp.where(lanes == l, ivec, 0))
                j = t * 16 + l
                pltpu.async_copy(ut_hbm.at[ui], urows_v.at[j], semu)
                pltpu.async_copy(it_hbm.at[ii], irows_v.at[j], semi)
            return carry

        lax.fori_loop(0, vecs_per_half, body, 0)
        hbase = base + h * HALF
        pltpu.make_async_copy(ue_hbm.at[pl.ds(hbase, HALF)], urows_v, semu).wait()
        pltpu.make_async_copy(ie_hbm.at[pl.ds(hbase, HALF)], irows_v, semi).wait()
        pltpu.sync_copy(urows_v, ue_hbm.at[pl.ds(hbase, HALF)])
        pltpu.sync_copy(irows_v, ie_hbm.at[pl.ds(hbase, HALF)])


@functools.cache
def _gather():
    return pl.kernel(
        _gather_body,
        out_type=(
            jax.ShapeDtypeStruct((BATCH, EMBED), jnp.float32),
            jax.ShapeDtypeStruct((BATCH, EMBED), jnp.float32),
        ),
        mesh=plsc.VectorSubcoreMesh(core_axis_name="c", subcore_axis_name="s",
                                    num_cores=NC, num_subcores=NS),
        scratch_types=[
            pltpu.VMEM((2 * NCHUNK, CHUNK), jnp.int32),
            pltpu.VMEM((HALF, EMBED), jnp.float32),
            pltpu.VMEM((HALF, EMBED), jnp.float32),
            pltpu.SemaphoreType.DMA,
            pltpu.SemaphoreType.DMA,
        ],
        compiler_params=pltpu.CompilerParams(needs_layout_passes=False),
    )


BLK = 1024


def _mlp_body(xu_ref, xi_ref, w1u_ref, w1i_ref, b1_ref, w2_ref, b2_ref,
              w3_ref, b3_ref, out_ref):
    x1 = jnp.dot(xu_ref[...], w1u_ref[...], preferred_element_type=jnp.float32)
    x2 = jnp.dot(xi_ref[...], w1i_ref[...], preferred_element_type=jnp.float32)
    h = jnp.maximum(x1 + x2 + b1_ref[...], 0.0)
    h = jnp.maximum(
        jnp.dot(h, w2_ref[...], preferred_element_type=jnp.float32) + b2_ref[...],
        0.0)
    out_ref[...] = (
        jnp.dot(h, w3_ref[...], preferred_element_type=jnp.float32) + b3_ref[...])


def _mlp(xu, xi, w1u, w1i, b1, w2, b2, w3, b3):
    grid = (BATCH // BLK,)
    full = lambda shape: pl.BlockSpec(shape, lambda i: (0,) * len(shape))
    return pl.pallas_call(
        _mlp_body,
        grid=grid,
        in_specs=[
            pl.BlockSpec((BLK, EMBED), lambda i: (i, 0)),
            pl.BlockSpec((BLK, EMBED), lambda i: (i, 0)),
            full((EMBED, 256)),
            full((EMBED, 256)),
            full((1, 256)),
            full((256, 64)),
            full((1, 64)),
            full((64, 1)),
            full((1, 1)),
        ],
        out_specs=pl.BlockSpec((BLK, 1), lambda i: (i, 0)),
        out_shape=jax.ShapeDtypeStruct((BATCH, 1), jnp.float32),
    )(xu, xi, w1u, w1i, b1, w2, b2, w3, b3)


def kernel(user_id, item_id, user_table, item_table, W1, b1, W2, b2, W3, b3):
    uid = user_id.astype(jnp.int32).reshape(NW, NCHUNK, CHUNK)
    iid = item_id.astype(jnp.int32).reshape(NW, NCHUNK, CHUNK)
    idx = jnp.concatenate([uid, iid], axis=1)
    ue, ie = _gather()(idx, user_table, item_table)
    return _mlp(ue, ie, W1[:EMBED], W1[EMBED:], b1.reshape(1, 256),
                W2, b2.reshape(1, 64), W3, b3.reshape(1, 1))
